# baseline (speedup 1.0000x reference)
import functools
import math

import jax
import jax.numpy as jnp
from jax.experimental import pallas as pl
from jax.experimental.pallas import tpu as pltpu
from jax.experimental.pallas import tpu_sc as plsc

N = 2048
D = 256
H = 8
HD = D // H
BQ = 256
GRID = N // BQ
RSCALE = 1.0 / math.sqrt(HD)
VR = 40


def _attn_body(x_blk, x_full, m_ref, wq, bq, wk, bk, wv, bv, woT, bo,
               gamma, beta, out_ref, kt_scr, vt_scr, m8):
    i = pl.program_id(0)
    f32 = jnp.float32
    bf16 = jnp.bfloat16

    f8 = jnp.float8_e4m3fn

    @pl.when(i == 0)
    def _():
        xb = x_full[...].astype(bf16)
        kt_scr[...] = (jax.lax.dot_general(
            wk[...], xb, (((1,), (1,)), ((), ())),
            preferred_element_type=f32) + bk[...]).astype(f8)
        vt = (jax.lax.dot_general(
            wv[...], xb, (((1,), (1,)), ((), ())),
            preferred_element_type=f32) + bv[...]).astype(f8)
        for h in range(H):
            vt_scr[h * VR:h * VR + HD, :] = vt[h * HD:(h + 1) * HD, :]
            vt_scr[h * VR + HD:h * VR + HD + 1, :] = jnp.ones((1, N), f8)
            vt_scr[h * VR + HD + 1:(h + 1) * VR, :] = (
                jnp.zeros((VR - HD - 1, N), f8))
        m8[...] = m_ref[...].astype(f32).astype(f8)

    xq = x_blk[...]
    qt = ((jax.lax.dot_general(wq[...], xq.astype(bf16),
                               (((1,), (1,)), ((), ())),
                               preferred_element_type=f32)
           + bq[...]) * RSCALE).astype(f8)

    mi = m8[pl.ds(i * BQ, BQ), :]
    reach = jax.lax.dot_general(mi, m8[...], (((1,), (0,)), ((), ())),
                                preferred_element_type=f32)
    nz16 = jnp.minimum(reach, 1.0).astype(bf16)

    att_rows = []
    for h in range(H):
        qh = qt[h * HD:(h + 1) * HD, :]
        kh = kt_scr[h * HD:(h + 1) * HD, :]
        s = jax.lax.dot_general(qh, kh, (((0,), (0,)), ((), ())),
                                preferred_element_type=f32)
        e8 = (jnp.exp(s.astype(bf16)) * nz16).astype(f8)
        vh = vt_scr[h * VR:h * VR + HD + 1, :]
        aT = jax.lax.dot_general(vh, e8, (((1,), (1,)), ((), ())),
                                 preferred_element_type=f32)
        att_rows.append(aT[:HD, :] / aT[HD:HD + 1, :])
    attT = jnp.concatenate(att_rows, axis=0)

    out = jax.lax.dot_general(attT.astype(bf16), woT[...],
                              (((0,), (0,)), ((), ())),
                              preferred_element_type=f32) + bo[...]
    y = out + xq
    mu = jnp.mean(y, axis=1, keepdims=True)
    yc = y - mu
    var = jnp.mean(yc * yc, axis=1, keepdims=True)
    out_ref[...] = yc * jax.lax.rsqrt(var + 1e-5) * gamma[...] + beta[...]


@jax.jit
def _attn_call(x, m, wq, bq, wk, bk, wv, bv, woT, bo, gamma, beta):
    full = lambda *_: (0, 0)
    specs = [
        pl.BlockSpec((BQ, D), lambda i: (i, 0)),
        pl.BlockSpec((N, D), full),
        pl.BlockSpec((N, N), full),
        pl.BlockSpec((D, D), full),
        pl.BlockSpec((D, 1), full),
        pl.BlockSpec((D, D), full),
        pl.BlockSpec((D, 1), full),
        pl.BlockSpec((D, D), full),
        pl.BlockSpec((D, 1), full),
        pl.BlockSpec((D, D), full),
        pl.BlockSpec((1, D), full),
        pl.BlockSpec((1, D), full),
        pl.BlockSpec((1, D), full),
    ]
    return pl.pallas_call(
        _attn_body,
        grid=(GRID,),
        in_specs=specs,
        out_specs=pl.BlockSpec((BQ, D), lambda i: (i, 0)),
        out_shape=jax.ShapeDtypeStruct((N, D), jnp.float32),
        scratch_shapes=[
            pltpu.VMEM((D, N), jnp.float8_e4m3fn),
            pltpu.VMEM((H * VR, N), jnp.float8_e4m3fn),
            pltpu.VMEM((N, N), jnp.float8_e4m3fn),
        ],
    )(x, x, m, wq, bq, wk, bk, wv, bv, woT, bo, gamma, beta)


E = 32768
NS = 16
EPW = E // NS
HALFW = N // 2 * N // 2
SLC = HALFW // NS
NIDX = 2 * EPW + 128
ZCH = 16384


def _sc_scatter_body(edges, out_hbm, sbuf, dbuf, idxbuf, valbuf, zbuf, shared):
    c = jax.lax.axis_index("c")
    sid = jax.lax.axis_index("s")
    lanes = jax.lax.iota(jnp.int32, 16)
    row0 = c * (N // 2)
    dump = jnp.int32(HALFW)
    lo = jnp.ones((16,), jnp.int32)
    hi = jnp.full((16,), 65536, jnp.int32)

    def _fill_zero(j, cr):
        zbuf[pl.ds(j * 16, 16)] = jnp.zeros((16,), jnp.int32)
        return cr
    jax.lax.fori_loop(0, ZCH // 16, _fill_zero, 0)
    for k in range(SLC // ZCH):
        pltpu.sync_copy(zbuf, shared.at[pl.ds(sid * SLC + k * ZCH, ZCH)])

    pltpu.sync_copy(edges.at[pl.ds(sid * EPW, EPW)], sbuf)
    pltpu.sync_copy(edges.at[pl.ds(E + sid * EPW, EPW)], dbuf)

    def _entry(off, a, b):
        rel = a - row0
        own = (rel >= 0) & (rel < N // 2)
        flat = rel * N + b
        idxbuf[pl.ds(off, 16)] = jnp.where(own, flat >> 1, dump)
        valbuf[pl.ds(off, 16)] = jnp.where((flat & 1) == 1, hi, lo)

    def _addr(j, cr):
        off = j * 16
        s = sbuf[pl.ds(off, 16)]
        d = dbuf[pl.ds(off, 16)]
        _entry(off, s, d)
        _entry(EPW + off, d, s)
        return cr
    jax.lax.fori_loop(0, EPW // 16, _addr, 0)

    for k in range(8):
        r = sid * 128 + k * 16 + lanes
        _entry(2 * EPW + k * 16, r, r)

    plsc.subcore_barrier()
    pltpu.sync_copy(valbuf, shared.at[idxbuf], add=True)
    plsc.subcore_barrier()

    pltpu.sync_copy(shared.at[pl.ds(sid * SLC, SLC)],
                    out_hbm.at[pl.ds(c * HALFW + sid * SLC, SLC)])


@jax.jit
def _build_m(edge_index):
    edges = edge_index.reshape(2 * E)
    call = pl.kernel(
        _sc_scatter_body,
        out_type=jax.ShapeDtypeStruct((N * N // 2,), jnp.int32),
        mesh=plsc.VectorSubcoreMesh(core_axis_name="c", subcore_axis_name="s"),
        compiler_params=pltpu.CompilerParams(needs_layout_passes=False),
        scratch_types=[
            pltpu.VMEM((EPW,), jnp.int32),
            pltpu.VMEM((EPW,), jnp.int32),
            pltpu.VMEM((NIDX,), jnp.int32),
            pltpu.VMEM((NIDX,), jnp.int32),
            pltpu.VMEM((ZCH,), jnp.int32),
            pltpu.VMEM_SHARED((HALFW + 16,), jnp.int32),
        ],
    )
    packed = call(edges)
    return jax.lax.bitcast_convert_type(
        packed.reshape(N, N // 2), jnp.int16).reshape(N, N)


def kernel(x, edge_index, Wq, bq, Wk, bk, Wv, bv, Wo, bo, gamma, beta):
    m = _build_m(edge_index.astype(jnp.int32))
    col = lambda b: b.reshape(D, 1)
    row = lambda b: b.reshape(1, D)
    return _attn_call(
        x, m,
        Wq.astype(jnp.bfloat16), col(bq),
        Wk.astype(jnp.bfloat16), col(bk),
        Wv.astype(jnp.bfloat16), col(bv),
        Wo.T.astype(jnp.bfloat16), row(bo),
        row(gamma), row(beta))


# speedup vs baseline: 1.6724x; 1.6724x over previous
import functools
import math

import jax
import jax.numpy as jnp
from jax.experimental import pallas as pl
from jax.experimental.pallas import tpu as pltpu
from jax.experimental.pallas import tpu_sc as plsc

N = 2048
D = 256
H = 8
HD = D // H
BQ = 256
GRID = N // BQ
RSCALE = 1.0 / math.sqrt(HD)
VR = 40


def _attn_body(x_blk, x_full, m_ref, wq, bq, wk, bk, wv, bv, woT, bo,
               gamma, beta, out_ref, kt_scr, vt_scr, m8):
    i = pl.program_id(0)
    f32 = jnp.float32
    bf16 = jnp.bfloat16

    f8 = jnp.float8_e4m3fn

    @pl.when(i == 0)
    def _():
        xb = x_full[...].astype(bf16)
        kt_scr[...] = (jax.lax.dot_general(
            wk[...], xb, (((1,), (1,)), ((), ())),
            preferred_element_type=f32) + bk[...]).astype(f8)
        vt = (jax.lax.dot_general(
            wv[...], xb, (((1,), (1,)), ((), ())),
            preferred_element_type=f32) + bv[...]).astype(f8)
        for h in range(H):
            vt_scr[h * VR:h * VR + HD, :] = vt[h * HD:(h + 1) * HD, :]
            vt_scr[h * VR + HD:h * VR + HD + 1, :] = jnp.ones((1, N), f8)
            vt_scr[h * VR + HD + 1:(h + 1) * VR, :] = (
                jnp.zeros((VR - HD - 1, N), f8))
        mi32 = m_ref[...]
        m8[:, :N // 2] = (mi32 & 0xffff).astype(f32).astype(f8)
        m8[:, N // 2:] = (mi32 >> 16).astype(f32).astype(f8)

    xq = x_blk[...]
    qt = ((jax.lax.dot_general(wq[...], xq.astype(bf16),
                               (((1,), (1,)), ((), ())),
                               preferred_element_type=f32)
           + bq[...]) * RSCALE).astype(f8)

    mi = m8[pl.ds(i * BQ, BQ), :]
    reach = jax.lax.dot_general(mi, m8[...], (((1,), (0,)), ((), ())),
                                preferred_element_type=f32)
    nz16 = jnp.minimum(reach, 1.0).astype(bf16)

    att_rows = []
    for h in range(H):
        qh = qt[h * HD:(h + 1) * HD, :]
        kh = kt_scr[h * HD:(h + 1) * HD, :]
        s = jax.lax.dot_general(qh, kh, (((0,), (0,)), ((), ())),
                                preferred_element_type=f32)
        e8 = (jnp.exp(s.astype(bf16)) * nz16).astype(f8)
        vh = vt_scr[h * VR:h * VR + HD + 1, :]
        aT = jax.lax.dot_general(vh, e8, (((1,), (1,)), ((), ())),
                                 preferred_element_type=f32)
        att_rows.append(aT[:HD, :] / aT[HD:HD + 1, :])
    attT = jnp.concatenate(att_rows, axis=0)

    out = jax.lax.dot_general(attT.astype(bf16), woT[...],
                              (((0,), (0,)), ((), ())),
                              preferred_element_type=f32) + bo[...]
    y = out + xq
    mu = jnp.mean(y, axis=1, keepdims=True)
    yc = y - mu
    var = jnp.mean(yc * yc, axis=1, keepdims=True)
    out_ref[...] = yc * jax.lax.rsqrt(var + 1e-5) * gamma[...] + beta[...]


@jax.jit
def _attn_call(x, m, wq, bq, wk, bk, wv, bv, woT, bo, gamma, beta):
    full = lambda *_: (0, 0)
    specs = [
        pl.BlockSpec((BQ, D), lambda i: (i, 0)),
        pl.BlockSpec((N, D), full),
        pl.BlockSpec((N, N // 2), full),
        pl.BlockSpec((D, D), full),
        pl.BlockSpec((D, 1), full),
        pl.BlockSpec((D, D), full),
        pl.BlockSpec((D, 1), full),
        pl.BlockSpec((D, D), full),
        pl.BlockSpec((D, 1), full),
        pl.BlockSpec((D, D), full),
        pl.BlockSpec((1, D), full),
        pl.BlockSpec((1, D), full),
        pl.BlockSpec((1, D), full),
    ]
    return pl.pallas_call(
        _attn_body,
        grid=(GRID,),
        in_specs=specs,
        out_specs=pl.BlockSpec((BQ, D), lambda i: (i, 0)),
        out_shape=jax.ShapeDtypeStruct((N, D), jnp.float32),
        scratch_shapes=[
            pltpu.VMEM((D, N), jnp.float8_e4m3fn),
            pltpu.VMEM((H * VR, N), jnp.float8_e4m3fn),
            pltpu.VMEM((N, N), jnp.float8_e4m3fn),
        ],
    )(x, x, m, wq, bq, wk, bk, wv, bv, woT, bo, gamma, beta)


E = 32768
NS = 16
EPW = E // NS
HALFW = N // 2 * N // 2
SLC = HALFW // NS
NIDX = 2 * EPW + 128
ZCH = 16384


def _sc_scatter_body(edges, out_hbm, sbuf, dbuf, idxbuf, valbuf, zbuf, shared):
    c = jax.lax.axis_index("c")
    sid = jax.lax.axis_index("s")
    lanes = jax.lax.iota(jnp.int32, 16)
    row0 = c * (N // 2)
    dump = jnp.int32(HALFW)
    lo = jnp.ones((16,), jnp.int32)
    hi = jnp.full((16,), 65536, jnp.int32)

    def _fill_zero(j, cr):
        zbuf[pl.ds(j * 16, 16)] = jnp.zeros((16,), jnp.int32)
        return cr
    jax.lax.fori_loop(0, ZCH // 16, _fill_zero, 0)
    for k in range(SLC // ZCH):
        pltpu.sync_copy(zbuf, shared.at[pl.ds(sid * SLC + k * ZCH, ZCH)])

    pltpu.sync_copy(edges.at[pl.ds(sid * EPW, EPW)], sbuf)
    pltpu.sync_copy(edges.at[pl.ds(E + sid * EPW, EPW)], dbuf)

    def _entry(off, a, b):
        rel = a - row0
        own = (rel >= 0) & (rel < N // 2)
        word = rel * (N // 2) + (b & (N // 2 - 1))
        idxbuf[pl.ds(off, 16)] = jnp.where(own, word, dump)
        valbuf[pl.ds(off, 16)] = jnp.where(b >= N // 2, hi, lo)

    def _addr(j, cr):
        off = j * 16
        s = sbuf[pl.ds(off, 16)]
        d = dbuf[pl.ds(off, 16)]
        _entry(off, s, d)
        _entry(EPW + off, d, s)
        return cr
    jax.lax.fori_loop(0, EPW // 16, _addr, 0)

    for k in range(8):
        r = sid * 128 + k * 16 + lanes
        _entry(2 * EPW + k * 16, r, r)

    plsc.subcore_barrier()
    pltpu.sync_copy(valbuf, shared.at[idxbuf], add=True)
    plsc.subcore_barrier()

    pltpu.sync_copy(shared.at[pl.ds(sid * SLC, SLC)],
                    out_hbm.at[pl.ds(c * HALFW + sid * SLC, SLC)])


@jax.jit
def _build_m(edge_index):
    edges = edge_index.reshape(2 * E)
    call = pl.kernel(
        _sc_scatter_body,
        out_type=jax.ShapeDtypeStruct((N * N // 2,), jnp.int32),
        mesh=plsc.VectorSubcoreMesh(core_axis_name="c", subcore_axis_name="s"),
        compiler_params=pltpu.CompilerParams(needs_layout_passes=False),
        scratch_types=[
            pltpu.VMEM((EPW,), jnp.int32),
            pltpu.VMEM((EPW,), jnp.int32),
            pltpu.VMEM((NIDX,), jnp.int32),
            pltpu.VMEM((NIDX,), jnp.int32),
            pltpu.VMEM((ZCH,), jnp.int32),
            pltpu.VMEM_SHARED((HALFW + 16,), jnp.int32),
        ],
    )
    return call(edges).reshape(N, N // 2)


def kernel(x, edge_index, Wq, bq, Wk, bk, Wv, bv, Wo, bo, gamma, beta):
    m = _build_m(edge_index.astype(jnp.int32))
    col = lambda b: b.reshape(D, 1)
    row = lambda b: b.reshape(1, D)
    return _attn_call(
        x, m,
        Wq.astype(jnp.bfloat16), col(bq),
        Wk.astype(jnp.bfloat16), col(bk),
        Wv.astype(jnp.bfloat16), col(bv),
        Wo.T.astype(jnp.bfloat16), row(bo),
        row(gamma), row(beta))


# speedup vs baseline: 1.7020x; 1.0177x over previous
import functools
import math

import jax
import jax.numpy as jnp
from jax.experimental import pallas as pl
from jax.experimental.pallas import tpu as pltpu
from jax.experimental.pallas import tpu_sc as plsc

N = 2048
D = 256
H = 8
HD = D // H
BQ = 512
GRID = N // BQ
RSCALE = 1.0 / math.sqrt(HD)
VR = 40


def _attn_body(x_blk, x_full, m_ref, wq, bq, wk, bk, wv, bv, woT, bo,
               gamma, beta, out_ref, kt_scr, vt_scr, m8):
    i = pl.program_id(0)
    f32 = jnp.float32
    bf16 = jnp.bfloat16

    f8 = jnp.float8_e4m3fn

    @pl.when(i == 0)
    def _():
        xb = x_full[...].astype(bf16)
        kt_scr[...] = (jax.lax.dot_general(
            wk[...], xb, (((1,), (1,)), ((), ())),
            preferred_element_type=f32) + bk[...]).astype(f8)
        vt = (jax.lax.dot_general(
            wv[...], xb, (((1,), (1,)), ((), ())),
            preferred_element_type=f32) + bv[...]).astype(f8)
        for h in range(H):
            vt_scr[h * VR:h * VR + HD, :] = vt[h * HD:(h + 1) * HD, :]
            vt_scr[h * VR + HD:h * VR + HD + 1, :] = jnp.ones((1, N), f8)
            vt_scr[h * VR + HD + 1:(h + 1) * VR, :] = (
                jnp.zeros((VR - HD - 1, N), f8))
        mi32 = m_ref[...]
        m8[:, :N // 2] = (mi32 & 0xffff).astype(f32).astype(f8)
        m8[:, N // 2:] = (mi32 >> 16).astype(f32).astype(f8)

    xq = x_blk[...]
    qt = ((jax.lax.dot_general(wq[...], xq.astype(bf16),
                               (((1,), (1,)), ((), ())),
                               preferred_element_type=f32)
           + bq[...]) * RSCALE).astype(f8)

    mi = m8[pl.ds(i * BQ, BQ), :]
    reach = jax.lax.dot_general(mi, m8[...], (((1,), (0,)), ((), ())),
                                preferred_element_type=f32)
    nz16 = jnp.minimum(reach, 1.0).astype(bf16)

    att_rows = []
    for h in range(H):
        qh = qt[h * HD:(h + 1) * HD, :]
        kh = kt_scr[h * HD:(h + 1) * HD, :]
        s = jax.lax.dot_general(qh, kh, (((0,), (0,)), ((), ())),
                                preferred_element_type=f32)
        e8 = (jnp.exp(s.astype(bf16)) * nz16).astype(f8)
        vh = vt_scr[h * VR:h * VR + HD + 1, :]
        aT = jax.lax.dot_general(vh, e8, (((1,), (1,)), ((), ())),
                                 preferred_element_type=f32)
        att_rows.append(aT[:HD, :] / aT[HD:HD + 1, :])
    attT = jnp.concatenate(att_rows, axis=0)

    out = jax.lax.dot_general(attT.astype(bf16), woT[...],
                              (((0,), (0,)), ((), ())),
                              preferred_element_type=f32) + bo[...]
    y = out + xq
    mu = jnp.mean(y, axis=1, keepdims=True)
    yc = y - mu
    var = jnp.mean(yc * yc, axis=1, keepdims=True)
    out_ref[...] = yc * jax.lax.rsqrt(var + 1e-5) * gamma[...] + beta[...]


@jax.jit
def _attn_call(x, m, wq, bq, wk, bk, wv, bv, woT, bo, gamma, beta):
    full = lambda *_: (0, 0)
    specs = [
        pl.BlockSpec((BQ, D), lambda i: (i, 0)),
        pl.BlockSpec((N, D), full),
        pl.BlockSpec((N, N // 2), full),
        pl.BlockSpec((D, D), full),
        pl.BlockSpec((D, 1), full),
        pl.BlockSpec((D, D), full),
        pl.BlockSpec((D, 1), full),
        pl.BlockSpec((D, D), full),
        pl.BlockSpec((D, 1), full),
        pl.BlockSpec((D, D), full),
        pl.BlockSpec((1, D), full),
        pl.BlockSpec((1, D), full),
        pl.BlockSpec((1, D), full),
    ]
    return pl.pallas_call(
        _attn_body,
        grid=(GRID,),
        in_specs=specs,
        out_specs=pl.BlockSpec((BQ, D), lambda i: (i, 0)),
        out_shape=jax.ShapeDtypeStruct((N, D), jnp.float32),
        scratch_shapes=[
            pltpu.VMEM((D, N), jnp.float8_e4m3fn),
            pltpu.VMEM((H * VR, N), jnp.float8_e4m3fn),
            pltpu.VMEM((N, N), jnp.float8_e4m3fn),
        ],
    )(x, x, m, wq, bq, wk, bk, wv, bv, woT, bo, gamma, beta)


E = 32768
NS = 16
EPW = E // NS
HALFW = N // 2 * N // 2
SLC = HALFW // NS
NIDX = 2 * EPW + 128
ZCH = 16384


def _sc_scatter_body(edges, out_hbm, sbuf, dbuf, idxbuf, valbuf, zbuf, shared):
    c = jax.lax.axis_index("c")
    sid = jax.lax.axis_index("s")
    lanes = jax.lax.iota(jnp.int32, 16)
    row0 = c * (N // 2)
    dump = jnp.int32(HALFW)
    lo = jnp.ones((16,), jnp.int32)
    hi = jnp.full((16,), 65536, jnp.int32)

    def _fill_zero(j, cr):
        zbuf[pl.ds(j * 16, 16)] = jnp.zeros((16,), jnp.int32)
        return cr
    jax.lax.fori_loop(0, ZCH // 16, _fill_zero, 0)
    for k in range(SLC // ZCH):
        pltpu.sync_copy(zbuf, shared.at[pl.ds(sid * SLC + k * ZCH, ZCH)])

    pltpu.sync_copy(edges.at[pl.ds(sid * EPW, EPW)], sbuf)
    pltpu.sync_copy(edges.at[pl.ds(E + sid * EPW, EPW)], dbuf)

    def _entry(off, a, b):
        rel = a - row0
        own = (rel >= 0) & (rel < N // 2)
        word = rel * (N // 2) + (b & (N // 2 - 1))
        idxbuf[pl.ds(off, 16)] = jnp.where(own, word, dump)
        valbuf[pl.ds(off, 16)] = jnp.where(b >= N // 2, hi, lo)

    def _addr(j, cr):
        off = j * 16
        s = sbuf[pl.ds(off, 16)]
        d = dbuf[pl.ds(off, 16)]
        _entry(off, s, d)
        _entry(EPW + off, d, s)
        return cr
    jax.lax.fori_loop(0, EPW // 16, _addr, 0)

    for k in range(8):
        r = sid * 128 + k * 16 + lanes
        _entry(2 * EPW + k * 16, r, r)

    plsc.subcore_barrier()
    pltpu.sync_copy(valbuf, shared.at[idxbuf], add=True)
    plsc.subcore_barrier()

    pltpu.sync_copy(shared.at[pl.ds(sid * SLC, SLC)],
                    out_hbm.at[pl.ds(c * HALFW + sid * SLC, SLC)])


@jax.jit
def _build_m(edge_index):
    edges = edge_index.reshape(2 * E)
    call = pl.kernel(
        _sc_scatter_body,
        out_type=jax.ShapeDtypeStruct((N * N // 2,), jnp.int32),
        mesh=plsc.VectorSubcoreMesh(core_axis_name="c", subcore_axis_name="s"),
        compiler_params=pltpu.CompilerParams(needs_layout_passes=False),
        scratch_types=[
            pltpu.VMEM((EPW,), jnp.int32),
            pltpu.VMEM((EPW,), jnp.int32),
            pltpu.VMEM((NIDX,), jnp.int32),
            pltpu.VMEM((NIDX,), jnp.int32),
            pltpu.VMEM((ZCH,), jnp.int32),
            pltpu.VMEM_SHARED((HALFW + 16,), jnp.int32),
        ],
    )
    return call(edges).reshape(N, N // 2)


def kernel(x, edge_index, Wq, bq, Wk, bk, Wv, bv, Wo, bo, gamma, beta):
    m = _build_m(edge_index.astype(jnp.int32))
    col = lambda b: b.reshape(D, 1)
    row = lambda b: b.reshape(1, D)
    return _attn_call(
        x, m,
        Wq.astype(jnp.bfloat16), col(bq),
        Wk.astype(jnp.bfloat16), col(bk),
        Wv.astype(jnp.bfloat16), col(bv),
        Wo.T.astype(jnp.bfloat16), row(bo),
        row(gamma), row(beta))


# speedup vs baseline: 1.8037x; 1.0598x over previous
import functools
import math

import jax
import jax.numpy as jnp
from jax.experimental import pallas as pl
from jax.experimental.pallas import tpu as pltpu
from jax.experimental.pallas import tpu_sc as plsc

N = 2048
D = 256
H = 8
HD = D // H
BQ = 512
GRID = N // BQ
RSCALE = 1.0 / math.sqrt(HD)
VR = 40


def _attn_body(x_blk, x_full, m_ref, wq, bq, wk, bk, wv, bv, woT, bo,
               gamma, beta, out_ref, kt_scr, vt_scr, m8):
    i = pl.program_id(0)
    f32 = jnp.float32
    bf16 = jnp.bfloat16

    f8 = jnp.float8_e4m3fn

    @pl.when(i == 0)
    def _():
        xb = x_full[...].astype(bf16)
        kt_scr[...] = (jax.lax.dot_general(
            wk[...], xb, (((1,), (1,)), ((), ())),
            preferred_element_type=f32) + bk[...]).astype(f8)
        vt = (jax.lax.dot_general(
            wv[...], xb, (((1,), (1,)), ((), ())),
            preferred_element_type=f32) + bv[...]).astype(f8)
        for h in range(H):
            vt_scr[h * VR:h * VR + HD, :] = vt[h * HD:(h + 1) * HD, :]
            vt_scr[h * VR + HD:h * VR + HD + 1, :] = jnp.ones((1, N), f8)
            vt_scr[h * VR + HD + 1:(h + 1) * VR, :] = (
                jnp.zeros((VR - HD - 1, N), f8))
        mi32 = m_ref[...]
        m8[:, :N // 2] = (mi32 & 0xffff).astype(f32).astype(f8)
        m8[:, N // 2:] = (mi32 >> 16).astype(f32).astype(f8)

    xq = x_blk[...]
    qt = ((jax.lax.dot_general(wq[...], xq.astype(bf16),
                               (((1,), (1,)), ((), ())),
                               preferred_element_type=f32)
           + bq[...]) * RSCALE).astype(f8)

    mi = m8[pl.ds(i * BQ, BQ), :]
    reach = jax.lax.dot_general(mi, m8[...], (((1,), (0,)), ((), ())),
                                preferred_element_type=f32)
    nz16 = jnp.minimum(reach, 1.0).astype(bf16)

    att_rows = []
    for h in range(H):
        qh = qt[h * HD:(h + 1) * HD, :]
        kh = kt_scr[h * HD:(h + 1) * HD, :]
        s = jax.lax.dot_general(qh, kh, (((0,), (0,)), ((), ())),
                                preferred_element_type=f32)
        e8 = (jnp.exp(s.astype(bf16)) * nz16).astype(f8)
        vh = vt_scr[h * VR:h * VR + HD + 1, :]
        aT = jax.lax.dot_general(vh, e8, (((1,), (1,)), ((), ())),
                                 preferred_element_type=f32)
        att_rows.append(aT[:HD, :] / aT[HD:HD + 1, :])
    attT = jnp.concatenate(att_rows, axis=0)

    out = jax.lax.dot_general(attT.astype(bf16), woT[...],
                              (((0,), (0,)), ((), ())),
                              preferred_element_type=f32) + bo[...]
    y = out + xq
    mu = jnp.mean(y, axis=1, keepdims=True)
    yc = y - mu
    var = jnp.mean(yc * yc, axis=1, keepdims=True)
    out_ref[...] = yc * jax.lax.rsqrt(var + 1e-5) * gamma[...] + beta[...]


@jax.jit
def _attn_call(x, m, wq, bq, wk, bk, wv, bv, woT, bo, gamma, beta):
    full = lambda *_: (0, 0)
    specs = [
        pl.BlockSpec((BQ, D), lambda i: (i, 0)),
        pl.BlockSpec((N, D), full),
        pl.BlockSpec((N, N // 2), full),
        pl.BlockSpec((D, D), full),
        pl.BlockSpec((D, 1), full),
        pl.BlockSpec((D, D), full),
        pl.BlockSpec((D, 1), full),
        pl.BlockSpec((D, D), full),
        pl.BlockSpec((D, 1), full),
        pl.BlockSpec((D, D), full),
        pl.BlockSpec((1, D), full),
        pl.BlockSpec((1, D), full),
        pl.BlockSpec((1, D), full),
    ]
    return pl.pallas_call(
        _attn_body,
        grid=(GRID,),
        in_specs=specs,
        out_specs=pl.BlockSpec((BQ, D), lambda i: (i, 0)),
        out_shape=jax.ShapeDtypeStruct((N, D), jnp.float32),
        scratch_shapes=[
            pltpu.VMEM((D, N), jnp.float8_e4m3fn),
            pltpu.VMEM((H * VR, N), jnp.float8_e4m3fn),
            pltpu.VMEM((N, N), jnp.float8_e4m3fn),
        ],
    )(x, x, m, wq, bq, wk, bk, wv, bv, woT, bo, gamma, beta)


E = 32768
NS = 16
EPW = E // NS
HALFW = N // 2 * N // 2
SLC = HALFW // NS
NIDX = 2 * EPW + 128
ZCH = 16384


def _sc_scatter_body(edges, out_hbm, sbuf, dbuf, idxbuf, valbuf, zbuf, shared,
                     esem, zsem):
    c = jax.lax.axis_index("c")
    sid = jax.lax.axis_index("s")
    lanes = jax.lax.iota(jnp.int32, 16)
    row0 = c * (N // 2)
    dump = jnp.int32(HALFW)
    lo = jnp.ones((16,), jnp.int32)
    hi = jnp.full((16,), 65536, jnp.int32)

    ecp1 = pltpu.async_copy(edges.at[pl.ds(sid * EPW, EPW)], sbuf, esem)
    ecp2 = pltpu.async_copy(edges.at[pl.ds(E + sid * EPW, EPW)], dbuf, esem)

    def _fill_zero(j, cr):
        for u in range(4):
            zbuf[pl.ds(j * 64 + u * 16, 16)] = jnp.zeros((16,), jnp.int32)
        return cr
    jax.lax.fori_loop(0, ZCH // 64, _fill_zero, 0)
    zcps = [pltpu.async_copy(zbuf, shared.at[pl.ds(sid * SLC + k * ZCH, ZCH)],
                             zsem)
            for k in range(SLC // ZCH)]
    ecp1.wait()
    ecp2.wait()

    def _entry(off, a, b):
        rel = a - row0
        own = (rel >= 0) & (rel < N // 2)
        word = rel * (N // 2) + (b & (N // 2 - 1))
        idxbuf[pl.ds(off, 16)] = jnp.where(own, word, dump)
        valbuf[pl.ds(off, 16)] = jnp.where(b >= N // 2, hi, lo)

    def _addr(j, cr):
        off = j * 16
        s = sbuf[pl.ds(off, 16)]
        d = dbuf[pl.ds(off, 16)]
        _entry(off, s, d)
        _entry(EPW + off, d, s)
        return cr
    jax.lax.fori_loop(0, EPW // 16, _addr, 0)

    for k in range(8):
        r = sid * 128 + k * 16 + lanes
        _entry(2 * EPW + k * 16, r, r)

    for z in zcps:
        z.wait()
    plsc.subcore_barrier()
    pltpu.sync_copy(valbuf, shared.at[idxbuf], add=True)
    plsc.subcore_barrier()

    pltpu.sync_copy(shared.at[pl.ds(sid * SLC, SLC)],
                    out_hbm.at[pl.ds(c * HALFW + sid * SLC, SLC)])


@jax.jit
def _build_m(edge_index):
    edges = edge_index.reshape(2 * E)
    call = pl.kernel(
        _sc_scatter_body,
        out_type=jax.ShapeDtypeStruct((N * N // 2,), jnp.int32),
        mesh=plsc.VectorSubcoreMesh(core_axis_name="c", subcore_axis_name="s"),
        compiler_params=pltpu.CompilerParams(needs_layout_passes=False),
        scratch_types=[
            pltpu.VMEM((EPW,), jnp.int32),
            pltpu.VMEM((EPW,), jnp.int32),
            pltpu.VMEM((NIDX,), jnp.int32),
            pltpu.VMEM((NIDX,), jnp.int32),
            pltpu.VMEM((ZCH,), jnp.int32),
            pltpu.VMEM_SHARED((HALFW + 16,), jnp.int32),
            pltpu.SemaphoreType.DMA,
            pltpu.SemaphoreType.DMA,
        ],
    )
    return call(edges).reshape(N, N // 2)


def kernel(x, edge_index, Wq, bq, Wk, bk, Wv, bv, Wo, bo, gamma, beta):
    m = _build_m(edge_index.astype(jnp.int32))
    col = lambda b: b.reshape(D, 1)
    row = lambda b: b.reshape(1, D)
    return _attn_call(
        x, m,
        Wq.astype(jnp.bfloat16), col(bq),
        Wk.astype(jnp.bfloat16), col(bk),
        Wv.astype(jnp.bfloat16), col(bv),
        Wo.T.astype(jnp.bfloat16), row(bo),
        row(gamma), row(beta))


# speedup vs baseline: 1.8729x; 1.0383x over previous
import functools
import math

import jax
import jax.numpy as jnp
from jax.experimental import pallas as pl
from jax.experimental.pallas import tpu as pltpu
from jax.experimental.pallas import tpu_sc as plsc

N = 2048
D = 256
H = 8
HD = D // H
BQ = 512
GRID = N // BQ
RSCALE = 1.0 / math.sqrt(HD)
VR = 40


def _qkv_body(x_ref, wq, bq, wk, bk, wv, bv, qt_out, kt_out, vt_out):
    f32 = jnp.float32
    f8 = jnp.float8_e4m3fn
    xb = x_ref[...].astype(jnp.bfloat16)
    qt_out[...] = ((jax.lax.dot_general(
        wq[...], xb, (((1,), (1,)), ((), ())),
        preferred_element_type=f32) + bq[...]) * RSCALE).astype(f8)
    kt_out[...] = (jax.lax.dot_general(
        wk[...], xb, (((1,), (1,)), ((), ())),
        preferred_element_type=f32) + bk[...]).astype(f8)
    vt = (jax.lax.dot_general(
        wv[...], xb, (((1,), (1,)), ((), ())),
        preferred_element_type=f32) + bv[...]).astype(f8)
    for h in range(H):
        vt_out[h * VR:h * VR + HD, :] = vt[h * HD:(h + 1) * HD, :]
        vt_out[h * VR + HD:h * VR + HD + 1, :] = jnp.ones((1, N), f8)
        vt_out[h * VR + HD + 1:(h + 1) * VR, :] = (
            jnp.zeros((VR - HD - 1, N), f8))


@jax.jit
def _qkv_call(x, wq, bq, wk, bk, wv, bv):
    full = lambda *_: (0, 0)
    f8 = jnp.float8_e4m3fn
    return pl.pallas_call(
        _qkv_body,
        in_specs=[
            pl.BlockSpec((N, D), full),
            pl.BlockSpec((D, D), full), pl.BlockSpec((D, 1), full),
            pl.BlockSpec((D, D), full), pl.BlockSpec((D, 1), full),
            pl.BlockSpec((D, D), full), pl.BlockSpec((D, 1), full),
        ],
        out_specs=[pl.BlockSpec((D, N), full),
                   pl.BlockSpec((D, N), full),
                   pl.BlockSpec((H * VR, N), full)],
        out_shape=[jax.ShapeDtypeStruct((D, N), f8),
                   jax.ShapeDtypeStruct((D, N), f8),
                   jax.ShapeDtypeStruct((H * VR, N), f8)],
    )(x, wq, bq, wk, bk, wv, bv)


def _attn_body(x_blk, m_ref, qt_ref, kt_scr, vt_scr, woT, bo,
               gamma, beta, out_ref, m8):
    i = pl.program_id(0)
    f32 = jnp.float32
    bf16 = jnp.bfloat16
    f8 = jnp.float8_e4m3fn

    @pl.when(i == 0)
    def _():
        mi32 = m_ref[...]
        m8[:, :N // 2] = (mi32 & 0xffff).astype(f32).astype(f8)
        m8[:, N // 2:] = (mi32 >> 16).astype(f32).astype(f8)

    xq = x_blk[...]
    qt = qt_ref[...]

    mi = m8[pl.ds(i * BQ, BQ), :]
    reach = jax.lax.dot_general(mi, m8[...], (((1,), (0,)), ((), ())),
                                preferred_element_type=f32)
    nz16 = jnp.minimum(reach, 1.0).astype(bf16)

    att_rows = []
    for h in range(H):
        qh = qt[h * HD:(h + 1) * HD, :]
        kh = kt_scr[h * HD:(h + 1) * HD, :]
        s = jax.lax.dot_general(qh, kh, (((0,), (0,)), ((), ())),
                                preferred_element_type=f32)
        e8 = (jnp.exp(s.astype(bf16)) * nz16).astype(f8)
        vh = vt_scr[h * VR:h * VR + HD + 1, :]
        aT = jax.lax.dot_general(vh, e8, (((1,), (1,)), ((), ())),
                                 preferred_element_type=f32)
        att_rows.append(aT[:HD, :] / aT[HD:HD + 1, :])
    attT = jnp.concatenate(att_rows, axis=0)

    out = jax.lax.dot_general(attT.astype(bf16), woT[...],
                              (((0,), (0,)), ((), ())),
                              preferred_element_type=f32) + bo[...]
    y = out + xq
    mu = jnp.mean(y, axis=1, keepdims=True)
    yc = y - mu
    var = jnp.mean(yc * yc, axis=1, keepdims=True)
    out_ref[...] = yc * jax.lax.rsqrt(var + 1e-5) * gamma[...] + beta[...]


@jax.jit
def _attn_call(x, m, qt, kt, vt, woT, bo, gamma, beta):
    full = lambda *_: (0, 0)
    specs = [
        pl.BlockSpec((BQ, D), lambda i: (i, 0)),
        pl.BlockSpec((N, N // 2), full),
        pl.BlockSpec((D, BQ), lambda i: (0, i)),
        pl.BlockSpec((D, N), full),
        pl.BlockSpec((H * VR, N), full),
        pl.BlockSpec((D, D), full),
        pl.BlockSpec((1, D), full),
        pl.BlockSpec((1, D), full),
        pl.BlockSpec((1, D), full),
    ]
    return pl.pallas_call(
        _attn_body,
        grid=(GRID,),
        in_specs=specs,
        out_specs=pl.BlockSpec((BQ, D), lambda i: (i, 0)),
        out_shape=jax.ShapeDtypeStruct((N, D), jnp.float32),
        scratch_shapes=[
            pltpu.VMEM((N, N), jnp.float8_e4m3fn),
        ],
    )(x, m, qt, kt, vt, woT, bo, gamma, beta)


E = 32768
NS = 16
EPW = E // NS
HALFW = N // 2 * N // 2
SLC = HALFW // NS
NIDX = 2 * EPW + 128
ZCH = 16384


def _sc_scatter_body(edges, out_hbm, sbuf, dbuf, idxbuf, valbuf, zbuf, shared,
                     esem, zsem):
    c = jax.lax.axis_index("c")
    sid = jax.lax.axis_index("s")
    lanes = jax.lax.iota(jnp.int32, 16)
    row0 = c * (N // 2)
    dump = jnp.int32(HALFW)
    lo = jnp.ones((16,), jnp.int32)
    hi = jnp.full((16,), 65536, jnp.int32)

    ecp1 = pltpu.async_copy(edges.at[pl.ds(sid * EPW, EPW)], sbuf, esem)
    ecp2 = pltpu.async_copy(edges.at[pl.ds(E + sid * EPW, EPW)], dbuf, esem)

    def _fill_zero(j, cr):
        for u in range(4):
            zbuf[pl.ds(j * 64 + u * 16, 16)] = jnp.zeros((16,), jnp.int32)
        return cr
    jax.lax.fori_loop(0, ZCH // 64, _fill_zero, 0)
    zcps = [pltpu.async_copy(zbuf, shared.at[pl.ds(sid * SLC + k * ZCH, ZCH)],
                             zsem)
            for k in range(SLC // ZCH)]
    ecp1.wait()
    ecp2.wait()

    def _entry(off, a, b):
        rel = a - row0
        own = (rel >= 0) & (rel < N // 2)
        word = rel * (N // 2) + (b & (N // 2 - 1))
        idxbuf[pl.ds(off, 16)] = jnp.where(own, word, dump)
        valbuf[pl.ds(off, 16)] = jnp.where(b >= N // 2, hi, lo)

    def _addr(j, cr):
        off = j * 16
        s = sbuf[pl.ds(off, 16)]
        d = dbuf[pl.ds(off, 16)]
        _entry(off, s, d)
        _entry(EPW + off, d, s)
        return cr
    jax.lax.fori_loop(0, EPW // 16, _addr, 0)

    for k in range(8):
        r = sid * 128 + k * 16 + lanes
        _entry(2 * EPW + k * 16, r, r)

    for z in zcps:
        z.wait()
    plsc.subcore_barrier()
    pltpu.sync_copy(valbuf, shared.at[idxbuf], add=True)
    plsc.subcore_barrier()

    pltpu.sync_copy(shared.at[pl.ds(sid * SLC, SLC)],
                    out_hbm.at[pl.ds(c * HALFW + sid * SLC, SLC)])


@jax.jit
def _build_m(edge_index):
    edges = edge_index.reshape(2 * E)
    call = pl.kernel(
        _sc_scatter_body,
        out_type=jax.ShapeDtypeStruct((N * N // 2,), jnp.int32),
        mesh=plsc.VectorSubcoreMesh(core_axis_name="c", subcore_axis_name="s"),
        compiler_params=pltpu.CompilerParams(needs_layout_passes=False),
        scratch_types=[
            pltpu.VMEM((EPW,), jnp.int32),
            pltpu.VMEM((EPW,), jnp.int32),
            pltpu.VMEM((NIDX,), jnp.int32),
            pltpu.VMEM((NIDX,), jnp.int32),
            pltpu.VMEM((ZCH,), jnp.int32),
            pltpu.VMEM_SHARED((HALFW + 16,), jnp.int32),
            pltpu.SemaphoreType.DMA,
            pltpu.SemaphoreType.DMA,
        ],
    )
    return call(edges).reshape(N, N // 2)


def kernel(x, edge_index, Wq, bq, Wk, bk, Wv, bv, Wo, bo, gamma, beta):
    m = _build_m(edge_index.astype(jnp.int32))
    col = lambda b: b.reshape(D, 1)
    row = lambda b: b.reshape(1, D)
    qt, kt, vt = _qkv_call(
        x,
        Wq.astype(jnp.bfloat16), col(bq),
        Wk.astype(jnp.bfloat16), col(bk),
        Wv.astype(jnp.bfloat16), col(bv))
    return _attn_call(
        x, m, qt, kt, vt,
        Wo.T.astype(jnp.bfloat16), row(bo),
        row(gamma), row(beta))


# speedup vs baseline: 1.8744x; 1.0008x over previous
import math

import jax
import jax.numpy as jnp
from jax.experimental import pallas as pl
from jax.experimental.pallas import tpu as pltpu
from jax.experimental.pallas import tpu_sc as plsc

N = 2048
D = 256
H = 8
HD = D // H
BQ = 512
GRID = N // BQ
RSCALE = 1.0 / math.sqrt(HD)
VR = 40


def _qkv_body(x_ref, wq, bq, wk, bk, wv, bv, qt_out, kt_out, vt_out):
    f32 = jnp.float32
    f8 = jnp.float8_e4m3fn
    xb = x_ref[...].astype(jnp.bfloat16)
    qt_out[...] = ((jax.lax.dot_general(
        wq[...], xb, (((1,), (1,)), ((), ())),
        preferred_element_type=f32) + bq[...]) * RSCALE).astype(f8)
    kt_out[...] = (jax.lax.dot_general(
        wk[...], xb, (((1,), (1,)), ((), ())),
        preferred_element_type=f32) + bk[...]).astype(f8)
    vt = (jax.lax.dot_general(
        wv[...], xb, (((1,), (1,)), ((), ())),
        preferred_element_type=f32) + bv[...]).astype(f8)
    for h in range(H):
        vt_out[h * VR:h * VR + HD, :] = vt[h * HD:(h + 1) * HD, :]
        vt_out[h * VR + HD:h * VR + HD + 1, :] = jnp.ones((1, N), f8)
        vt_out[h * VR + HD + 1:(h + 1) * VR, :] = (
            jnp.zeros((VR - HD - 1, N), f8))


@jax.jit
def _qkv_call(x, wq, bq, wk, bk, wv, bv):
    full = lambda *_: (0, 0)
    f8 = jnp.float8_e4m3fn
    return pl.pallas_call(
        _qkv_body,
        in_specs=[
            pl.BlockSpec((N, D), full),
            pl.BlockSpec((D, D), full), pl.BlockSpec((D, 1), full),
            pl.BlockSpec((D, D), full), pl.BlockSpec((D, 1), full),
            pl.BlockSpec((D, D), full), pl.BlockSpec((D, 1), full),
        ],
        out_specs=[pl.BlockSpec((D, N), full),
                   pl.BlockSpec((D, N), full),
                   pl.BlockSpec((H * VR, N), full)],
        out_shape=[jax.ShapeDtypeStruct((D, N), f8),
                   jax.ShapeDtypeStruct((D, N), f8),
                   jax.ShapeDtypeStruct((H * VR, N), f8)],
    )(x, wq, bq, wk, bk, wv, bv)


def _attn_body(x_blk, m_ref, qt_ref, kt_scr, vt_scr, woT, bo,
               gamma, beta, out_ref, m8):
    i = pl.program_id(0)
    f32 = jnp.float32
    bf16 = jnp.bfloat16
    f8 = jnp.float8_e4m3fn

    @pl.when(i == 0)
    def _():
        mi32 = m_ref[...]
        m8[:, :N // 2] = (mi32 & 0xffff).astype(f32).astype(f8)
        m8[:, N // 2:] = (mi32 >> 16).astype(f32).astype(f8)

    xq = x_blk[...]
    qt = qt_ref[...]

    mi = m8[pl.ds(i * BQ, BQ), :]
    reach = jax.lax.dot_general(mi, m8[...], (((1,), (0,)), ((), ())),
                                preferred_element_type=f32)
    nz16 = jnp.minimum(reach, 1.0).astype(bf16)

    att_rows = []
    for h in range(H):
        qh = qt[h * HD:(h + 1) * HD, :]
        kh = kt_scr[h * HD:(h + 1) * HD, :]
        s = jax.lax.dot_general(qh, kh, (((0,), (0,)), ((), ())),
                                preferred_element_type=f32)
        e8 = (jnp.exp(s.astype(bf16)) * nz16).astype(f8)
        vh = vt_scr[h * VR:h * VR + HD + 1, :]
        aT = jax.lax.dot_general(vh, e8, (((1,), (1,)), ((), ())),
                                 preferred_element_type=f32)
        att_rows.append(aT[:HD, :] / aT[HD:HD + 1, :])
    attT = jnp.concatenate(att_rows, axis=0)

    out = jax.lax.dot_general(attT.astype(bf16), woT[...],
                              (((0,), (0,)), ((), ())),
                              preferred_element_type=f32) + bo[...]
    y = out + xq
    mu = jnp.mean(y, axis=1, keepdims=True)
    yc = y - mu
    var = jnp.mean(yc * yc, axis=1, keepdims=True)
    out_ref[...] = yc * jax.lax.rsqrt(var + 1e-5) * gamma[...] + beta[...]


@jax.jit
def _attn_call(x, m, qt, kt, vt, woT, bo, gamma, beta):
    full = lambda *_: (0, 0)
    specs = [
        pl.BlockSpec((BQ, D), lambda i: (i, 0)),
        pl.BlockSpec((N, N // 2), full),
        pl.BlockSpec((D, BQ), lambda i: (0, i)),
        pl.BlockSpec((D, N), full),
        pl.BlockSpec((H * VR, N), full),
        pl.BlockSpec((D, D), full),
        pl.BlockSpec((1, D), full),
        pl.BlockSpec((1, D), full),
        pl.BlockSpec((1, D), full),
    ]
    return pl.pallas_call(
        _attn_body,
        grid=(GRID,),
        in_specs=specs,
        out_specs=pl.BlockSpec((BQ, D), lambda i: (i, 0)),
        out_shape=jax.ShapeDtypeStruct((N, D), jnp.float32),
        scratch_shapes=[
            pltpu.VMEM((N, N), jnp.float8_e4m3fn),
        ],
    )(x, m, qt, kt, vt, woT, bo, gamma, beta)


E = 32768
NS = 16
EPW = E // NS
HALFW = N // 2 * N // 2
SLC = HALFW // NS
NIDX = 2 * EPW + 128
ZCH = 16384


def _sc_scatter_body(edges, out_hbm, sbuf, dbuf, idxbuf, valbuf, zbuf, shared,
                     esem, zsem):
    c = jax.lax.axis_index("c")
    sid = jax.lax.axis_index("s")
    lanes = jax.lax.iota(jnp.int32, 16)
    row0 = c * (N // 2)
    dump = jnp.int32(HALFW)
    lo = jnp.ones((16,), jnp.int32)
    hi = jnp.full((16,), 65536, jnp.int32)

    ecp1 = pltpu.async_copy(edges.at[pl.ds(sid * EPW, EPW)], sbuf, esem)
    ecp2 = pltpu.async_copy(edges.at[pl.ds(E + sid * EPW, EPW)], dbuf, esem)

    def _fill_zero(j, cr):
        for u in range(4):
            zbuf[pl.ds(j * 64 + u * 16, 16)] = jnp.zeros((16,), jnp.int32)
        return cr
    jax.lax.fori_loop(0, ZCH // 64, _fill_zero, 0)
    zcps = [pltpu.async_copy(zbuf, shared.at[pl.ds(sid * SLC + k * ZCH, ZCH)],
                             zsem)
            for k in range(SLC // ZCH)]
    ecp1.wait()
    ecp2.wait()

    def _entry(off, a, b):
        rel = a - row0
        own = (rel >= 0) & (rel < N // 2)
        word = rel * (N // 2) + (b & (N // 2 - 1))
        idxbuf[pl.ds(off, 16)] = jnp.where(own, word, dump)
        valbuf[pl.ds(off, 16)] = jnp.where(b >= N // 2, hi, lo)

    def _addr(j, cr):
        off = j * 16
        s = sbuf[pl.ds(off, 16)]
        d = dbuf[pl.ds(off, 16)]
        _entry(off, s, d)
        _entry(EPW + off, d, s)
        return cr
    jax.lax.fori_loop(0, EPW // 16, _addr, 0)

    for k in range(8):
        r = sid * 128 + k * 16 + lanes
        _entry(2 * EPW + k * 16, r, r)

    for z in zcps:
        z.wait()
    plsc.subcore_barrier()
    pltpu.sync_copy(valbuf, shared.at[idxbuf], add=True)
    plsc.subcore_barrier()

    pltpu.sync_copy(shared.at[pl.ds(sid * SLC, SLC)],
                    out_hbm.at[pl.ds(c * HALFW + sid * SLC, SLC)])


@jax.jit
def _build_m(edge_index):
    edges = edge_index.reshape(2 * E)
    call = pl.kernel(
        _sc_scatter_body,
        out_type=jax.ShapeDtypeStruct((N * N // 2,), jnp.int32),
        mesh=plsc.VectorSubcoreMesh(core_axis_name="c", subcore_axis_name="s"),
        compiler_params=pltpu.CompilerParams(needs_layout_passes=False),
        scratch_types=[
            pltpu.VMEM((EPW,), jnp.int32),
            pltpu.VMEM((EPW,), jnp.int32),
            pltpu.VMEM((NIDX,), jnp.int32),
            pltpu.VMEM((NIDX,), jnp.int32),
            pltpu.VMEM((ZCH,), jnp.int32),
            pltpu.VMEM_SHARED((HALFW + 16,), jnp.int32),
            pltpu.SemaphoreType.DMA,
            pltpu.SemaphoreType.DMA,
        ],
    )
    return call(edges).reshape(N, N // 2)


def kernel(x, edge_index, Wq, bq, Wk, bk, Wv, bv, Wo, bo, gamma, beta):
    m = _build_m(edge_index.astype(jnp.int32))
    col = lambda b: b.reshape(D, 1)
    row = lambda b: b.reshape(1, D)
    qt, kt, vt = _qkv_call(
        x,
        Wq.astype(jnp.bfloat16), col(bq),
        Wk.astype(jnp.bfloat16), col(bk),
        Wv.astype(jnp.bfloat16), col(bv))
    return _attn_call(
        x, m, qt, kt, vt,
        Wo.T.astype(jnp.bfloat16), row(bo),
        row(gamma), row(beta))
